# baseline (speedup 1.0000x reference)
import functools

import jax
import jax.numpy as jnp
from jax import lax
from jax.experimental import pallas as pl
from jax.experimental.pallas import tpu as pltpu
from jax.experimental.pallas import tpu_sc as plsc

BS = 4
N = 2048
LS = 32
KS = 16
IN_CH = 128
OUT_CH = 128
XW = 16
NBR = LS - 1
TOT = BS * N * LS

ROWS1 = 256


def _topk_body(xyz_all_ref, q_ref, idx_ref):
    b = pl.program_id(0)
    t = pl.program_id(1)
    x = xyz_all_ref[...]
    q = q_ref[...]
    qn = jnp.sum(q * q, axis=-1)
    xn = jnp.sum(x * x, axis=-1)
    qx = lax.dot_general(q, x, (((1,), (1,)), ((), ())),
                         preferred_element_type=jnp.float32)
    d = qn[:, None] + xn[None, :] - 2.0 * qx
    d = jnp.where(d < 1e-8, jnp.inf, d)
    colf = lax.broadcasted_iota(jnp.int32, (ROWS1, N), 1).astype(jnp.float32)
    base = b * N
    row0 = t * ROWS1
    self_idx = base + row0 + lax.broadcasted_iota(jnp.int32, (ROWS1,), 0)
    idx_ref[:, 0] = self_idx
    nf = jnp.float32(N)
    for k in range(NBR):
        m = jnp.min(d, axis=1, keepdims=True)
        amf = jnp.min(jnp.where(d == m, colf, nf), axis=1)
        idx_ref[:, k + 1] = amf.astype(jnp.int32) + base
        d = jnp.where(colf == amf[:, None], jnp.inf, d)


def _topk(xyz):
    return pl.pallas_call(
        _topk_body,
        grid=(BS, N // ROWS1),
        in_specs=[
            pl.BlockSpec((None, N, 3), lambda b, t: (b, 0, 0)),
            pl.BlockSpec((None, ROWS1, 3), lambda b, t: (b, t, 0)),
        ],
        out_specs=pl.BlockSpec((None, ROWS1, LS), lambda b, t: (b, t, 0)),
        out_shape=jax.ShapeDtypeStruct((BS, N, LS), jnp.int32),
    )(xyz, xyz)


NC = 2
NS = 16
NW = NC * NS
PER_W = TOT // NW
CH = 512
NCHUNK = PER_W // CH


def _gather_body(table_d, table_x, idx_hbm, out_d, out_x,
                 ia, ib, da, db, xa, xb,
                 gd0, gd1, gx0, gx1, wd0, wd1, wx0, wx1):
    wid = lax.axis_index("s") * NC + lax.axis_index("c")
    base = wid * PER_W
    idxb = [ia, ib]
    dbuf = [da, db]
    xbuf = [xa, xb]
    gds = [gd0, gd1]
    gxs = [gx0, gx1]
    wds = [wd0, wd1]
    wxs = [wx0, wx1]

    def point_rel_all(buf):
        def point_rel(p, c2):
            r0 = p * LS
            self_v = buf[r0, :]
            for l in range(1, LS):
                buf[r0 + l, :] = buf[r0 + l, :] - self_v
            buf[r0, :] = self_v - self_v
            return c2

        lax.fori_loop(0, CH // LS, point_rel, 0)

    def chunk(ci, carry):
        off = base + ci * CH
        pltpu.sync_copy(idx_hbm.at[pl.ds(off, CH)], ia)
        cp_d = pltpu.async_copy(table_d.at[ia], da, gd0)
        cp_x = pltpu.async_copy(table_x.at[ia], xa, gx0)
        cp_x.wait()
        point_rel_all(xa)
        cp_d.wait()
        pltpu.sync_copy(da, out_d.at[pl.ds(off, CH)])
        pltpu.sync_copy(xa, out_x.at[pl.ds(off, CH)])
        return carry

    lax.fori_loop(0, NCHUNK, chunk, 0)


def _gather(table_d, table_x, idx_flat):
    mesh = plsc.VectorSubcoreMesh(core_axis_name="c", subcore_axis_name="s")
    k = functools.partial(
        pl.kernel,
        mesh=mesh,
        compiler_params=pltpu.CompilerParams(use_tc_tiling_on_sc=False),
        out_type=[
            jax.ShapeDtypeStruct((TOT, IN_CH), jnp.bfloat16),
            jax.ShapeDtypeStruct((TOT, XW), jnp.float32),
        ],
        scratch_types=[
            pltpu.VMEM((CH,), jnp.int32),
            pltpu.VMEM((CH,), jnp.int32),
            pltpu.VMEM((CH, IN_CH), jnp.bfloat16),
            pltpu.VMEM((CH, IN_CH), jnp.bfloat16),
            pltpu.VMEM((CH, XW), jnp.float32),
            pltpu.VMEM((CH, XW), jnp.float32),
        ] + [pltpu.SemaphoreType.DMA] * 8,
    )(_gather_body)
    return k(table_d, table_x, idx_flat)


PTS3 = 512
NT3 = BS * N // PTS3


def _moments_body(lx_ref, s1_ref, s2_ref):
    step = pl.program_id(0)
    r2 = lx_ref[...]
    p1 = jnp.sum(r2, axis=0)[None, :]
    p2 = lax.dot_general(r2, r2, (((0,), (0,)), ((), ())),
                         preferred_element_type=jnp.float32,
                         precision=lax.Precision.HIGHEST)

    @pl.when(step == 0)
    def _():
        s1_ref[...] = p1
        s2_ref[...] = p2

    @pl.when(step != 0)
    def _():
        s1_ref[...] += p1
        s2_ref[...] += p2


def _moments(rel2):
    return pl.pallas_call(
        _moments_body,
        grid=(NT3,),
        in_specs=[pl.BlockSpec((PTS3 * LS, XW), lambda t: (t, 0))],
        out_specs=[
            pl.BlockSpec((1, XW), lambda t: (0, 0)),
            pl.BlockSpec((XW, XW), lambda t: (0, 0)),
        ],
        out_shape=[
            jax.ShapeDtypeStruct((1, XW), jnp.float32),
            jax.ShapeDtypeStruct((XW, XW), jnp.float32),
        ],
    )(rel2)


PTS4 = 512
NT4 = BS * N // PTS4


def _alpha_body(rel_ref, s1_ref, s2_ref, w1_ref, b1_ref, g_ref, be_ref,
                w2_ref, b2_ref, alpha_ref):
    hi = lax.Precision.DEFAULT
    ntot = jnp.float32(TOT)
    mu = s1_ref[0, :] / ntot
    m2 = s2_ref[...] / ntot - mu[:, None] * mu[None, :]
    w1 = w1_ref[...]
    mean_h = jnp.sum(w1 * mu[None, :], axis=1) + b1_ref[0, :]
    wm = lax.dot_general(w1, m2, (((1,), (0,)), ((), ())),
                         preferred_element_type=jnp.float32,
                         precision=lax.Precision.HIGHEST)
    var_h = jnp.sum(wm * w1, axis=1)
    scale = g_ref[0, :] / jnp.sqrt(var_h + 1e-5)
    beff = (b1_ref[0, :] - mean_h) * scale + be_ref[0, :]
    weff = w1 * scale[:, None]

    rel = rel_ref[...]
    h = lax.dot_general(rel, weff, (((1,), (1,)), ((), ())),
                        preferred_element_type=jnp.float32,
                        precision=hi) + beff[None, :]
    h = jnp.maximum(h, 0.0)
    logits = lax.dot_general(h, w2_ref[...], (((1,), (1,)), ((), ())),
                             preferred_element_type=jnp.float32,
                             precision=hi)
    logits = logits + b2_ref[0, :][None, :]
    m = jnp.max(logits, axis=1, keepdims=True)
    e = jnp.exp(logits - m)
    alpha_ref[...] = e / jnp.sum(e, axis=1, keepdims=True)


def _alpha(rel2, s1, s2, w1p, b1, gamma, beta, w2, b2):
    return pl.pallas_call(
        _alpha_body,
        grid=(NT4,),
        in_specs=[
            pl.BlockSpec((PTS4 * LS, XW), lambda t: (t, 0)),
            pl.BlockSpec((1, XW), lambda t: (0, 0)),
            pl.BlockSpec((XW, XW), lambda t: (0, 0)),
            pl.BlockSpec((32, XW), lambda t: (0, 0)),
            pl.BlockSpec((1, 32), lambda t: (0, 0)),
            pl.BlockSpec((1, 32), lambda t: (0, 0)),
            pl.BlockSpec((1, 32), lambda t: (0, 0)),
            pl.BlockSpec((KS, 32), lambda t: (0, 0)),
            pl.BlockSpec((1, KS), lambda t: (0, 0)),
        ],
        out_specs=pl.BlockSpec((PTS4 * LS, KS), lambda t: (t, 0)),
        out_shape=jax.ShapeDtypeStruct((TOT, KS), jnp.float32),
    )(rel2, s1, s2, w1p, b1, gamma, beta, w2, b2)


PTS5 = 256
NT5 = BS * N // PTS5


def _bmmconv_body(ld_ref, alpha_ref, cw_ref, cb_ref, out_ref):
    hi = lax.Precision.DEFAULT
    a3 = alpha_ref[...].reshape(PTS5, LS, KS).astype(jnp.bfloat16)
    ld = ld_ref[...]
    g = lax.dot_general(a3, ld, (((1,), (1,)), ((0,), (0,))),
                        preferred_element_type=jnp.float32,
                        precision=hi)
    acc = jnp.zeros((PTS5, OUT_CH), jnp.float32) + cb_ref[0, :][None, :]
    for k in range(KS):
        acc = acc + lax.dot_general(g[:, k, :], cw_ref[k],
                                    (((1,), (0,)), ((), ())),
                                    preferred_element_type=jnp.float32,
                                    precision=hi)
    out_ref[...] = acc


def _bmmconv(local_d3, alpha, conv_kco, conv_b):
    return pl.pallas_call(
        _bmmconv_body,
        grid=(NT5,),
        in_specs=[
            pl.BlockSpec((PTS5, LS, IN_CH), lambda t: (t, 0, 0)),
            pl.BlockSpec((PTS5 * LS, KS), lambda t: (t, 0)),
            pl.BlockSpec((KS, IN_CH, OUT_CH), lambda t: (0, 0, 0)),
            pl.BlockSpec((1, OUT_CH), lambda t: (0, 0)),
        ],
        out_specs=pl.BlockSpec((PTS5, OUT_CH), lambda t: (t, 0)),
        out_shape=jax.ShapeDtypeStruct((BS * N, OUT_CH), jnp.float32),
    )(local_d3, alpha, conv_kco, conv_b)


def kernel(xyz, data, W1, b1, gamma, beta, W2, b2, conv_w, conv_b):
    idx = _topk(xyz)

    table_d = data.astype(jnp.bfloat16).reshape(BS * N, IN_CH)
    pad = jnp.zeros((BS, N, XW - 3), jnp.float32)
    table_x = jnp.concatenate([xyz, pad], axis=-1).reshape(BS * N, XW)
    local_d, rel2 = _gather(table_d, table_x, idx.reshape(TOT))
    local_d3 = local_d.reshape(BS * N, LS, IN_CH)

    s1, s2 = _moments(rel2)

    w1p = jnp.concatenate([W1, jnp.zeros((32, XW - 3), jnp.float32)], axis=1)
    alpha = _alpha(rel2, s1, s2, w1p, b1[None, :], gamma[None, :],
                   beta[None, :], W2, b2[None, :])

    conv_kco = conv_w.transpose(2, 1, 0)
    out = _bmmconv(local_d3, alpha, conv_kco, conv_b[None, :])
    return (xyz, out.reshape(BS, N, OUT_CH))


# speedup vs baseline: 1.2324x; 1.2324x over previous
import functools

import jax
import jax.numpy as jnp
from jax import lax
from jax.experimental import pallas as pl
from jax.experimental.pallas import tpu as pltpu
from jax.experimental.pallas import tpu_sc as plsc

BS = 4
N = 2048
LS = 32
KS = 16
IN_CH = 128
OUT_CH = 128
XW = 16
NBR = LS - 1
TOT = BS * N * LS

ROWS1 = 256


def _topk_body(xyz_all_ref, q_ref, idx_ref):
    b = pl.program_id(0)
    t = pl.program_id(1)
    x = xyz_all_ref[...]
    q = q_ref[...]
    qn = jnp.sum(q * q, axis=-1)
    xn = jnp.sum(x * x, axis=-1)
    qx = lax.dot_general(q, x, (((1,), (1,)), ((), ())),
                         preferred_element_type=jnp.float32)
    d = qn[:, None] + xn[None, :] - 2.0 * qx
    d = jnp.where(d < 1e-8, jnp.inf, d)
    colf = lax.broadcasted_iota(jnp.int32, (ROWS1, N), 1).astype(jnp.float32)
    base = b * N
    row0 = t * ROWS1
    self_idx = base + row0 + lax.broadcasted_iota(jnp.int32, (ROWS1,), 0)
    idx_ref[:, 0] = self_idx
    nf = jnp.float32(N)
    for k in range(NBR):
        m = jnp.min(d, axis=1, keepdims=True)
        amf = jnp.min(jnp.where(d == m, colf, nf), axis=1)
        idx_ref[:, k + 1] = amf.astype(jnp.int32) + base
        d = jnp.where(colf == amf[:, None], jnp.inf, d)


def _topk(xyz):
    return pl.pallas_call(
        _topk_body,
        grid=(BS, N // ROWS1),
        in_specs=[
            pl.BlockSpec((None, N, 3), lambda b, t: (b, 0, 0)),
            pl.BlockSpec((None, ROWS1, 3), lambda b, t: (b, t, 0)),
        ],
        out_specs=pl.BlockSpec((None, ROWS1, LS), lambda b, t: (b, t, 0)),
        out_shape=jax.ShapeDtypeStruct((BS, N, LS), jnp.int32),
    )(xyz, xyz)


NC = 2
NS = 16
NW = NC * NS
PER_W = TOT // NW
CH = 512
NCHUNK = PER_W // CH


def _gather_body(table_d, table_x, idx_hbm, out_d, out_x,
                 ia, ib, da, db, xa, xb,
                 gd0, gd1, gx0, gx1, wd0, wd1, wx0, wx1):
    wid = lax.axis_index("s") * NC + lax.axis_index("c")
    base = wid * PER_W
    idxb = [ia, ib]
    dbuf = [da, db]
    xbuf = [xa, xb]
    gds = [gd0, gd1]
    gxs = [gx0, gx1]
    wds = [wd0, wd1]
    wxs = [wx0, wx1]

    def point_rel_all(buf):
        def point_rel(p, c2):
            r0 = p * LS
            self_v = buf[r0, :]
            for l in range(1, LS):
                buf[r0 + l, :] = buf[r0 + l, :] - self_v
            buf[r0, :] = self_v - self_v
            return c2

        lax.fori_loop(0, CH // LS, point_rel, 0)

    def chunk(ci, carry):
        off = base + ci * CH
        pltpu.sync_copy(idx_hbm.at[pl.ds(off, CH)], ia)
        cp_d = pltpu.async_copy(table_d.at[ia], da, gd0)
        cp_x = pltpu.async_copy(table_x.at[ia], xa, gx0)
        cp_x.wait()
        point_rel_all(xa)
        cp_d.wait()
        pltpu.sync_copy(da, out_d.at[pl.ds(off, CH)])
        pltpu.sync_copy(xa, out_x.at[pl.ds(off, CH)])
        return carry

    lax.fori_loop(0, NCHUNK, chunk, 0)


def _gather(table_d, table_x, idx_flat):
    mesh = plsc.VectorSubcoreMesh(core_axis_name="c", subcore_axis_name="s")
    k = functools.partial(
        pl.kernel,
        mesh=mesh,
        compiler_params=pltpu.CompilerParams(use_tc_tiling_on_sc=False),
        out_type=[
            jax.ShapeDtypeStruct((TOT, IN_CH), jnp.float32),
            jax.ShapeDtypeStruct((TOT, XW), jnp.float32),
        ],
        scratch_types=[
            pltpu.VMEM((CH,), jnp.int32),
            pltpu.VMEM((CH,), jnp.int32),
            pltpu.VMEM((CH, IN_CH), jnp.float32),
            pltpu.VMEM((CH, IN_CH), jnp.float32),
            pltpu.VMEM((CH, XW), jnp.float32),
            pltpu.VMEM((CH, XW), jnp.float32),
        ] + [pltpu.SemaphoreType.DMA] * 8,
    )(_gather_body)
    return k(table_d, table_x, idx_flat)


PTS3 = 512
NT3 = BS * N // PTS3


def _moments_body(lx_ref, s1_ref, s2_ref):
    step = pl.program_id(0)
    r2 = lx_ref[...]
    p1 = jnp.sum(r2, axis=0)[None, :]
    p2 = lax.dot_general(r2, r2, (((0,), (0,)), ((), ())),
                         preferred_element_type=jnp.float32,
                         precision=lax.Precision.HIGHEST)

    @pl.when(step == 0)
    def _():
        s1_ref[...] = p1
        s2_ref[...] = p2

    @pl.when(step != 0)
    def _():
        s1_ref[...] += p1
        s2_ref[...] += p2


def _moments(rel2):
    return pl.pallas_call(
        _moments_body,
        grid=(NT3,),
        in_specs=[pl.BlockSpec((PTS3 * LS, XW), lambda t: (t, 0))],
        out_specs=[
            pl.BlockSpec((1, XW), lambda t: (0, 0)),
            pl.BlockSpec((XW, XW), lambda t: (0, 0)),
        ],
        out_shape=[
            jax.ShapeDtypeStruct((1, XW), jnp.float32),
            jax.ShapeDtypeStruct((XW, XW), jnp.float32),
        ],
    )(rel2)


PTS4 = 512
NT4 = BS * N // PTS4


def _alpha_body(rel_ref, s1_ref, s2_ref, w1_ref, b1_ref, g_ref, be_ref,
                w2_ref, b2_ref, alpha_ref):
    hi = lax.Precision.DEFAULT
    ntot = jnp.float32(TOT)
    mu = s1_ref[0, :] / ntot
    m2 = s2_ref[...] / ntot - mu[:, None] * mu[None, :]
    w1 = w1_ref[...]
    mean_h = jnp.sum(w1 * mu[None, :], axis=1) + b1_ref[0, :]
    wm = lax.dot_general(w1, m2, (((1,), (0,)), ((), ())),
                         preferred_element_type=jnp.float32,
                         precision=lax.Precision.HIGHEST)
    var_h = jnp.sum(wm * w1, axis=1)
    scale = g_ref[0, :] / jnp.sqrt(var_h + 1e-5)
    beff = (b1_ref[0, :] - mean_h) * scale + be_ref[0, :]
    weff = w1 * scale[:, None]

    rel = rel_ref[...]
    h = lax.dot_general(rel, weff, (((1,), (1,)), ((), ())),
                        preferred_element_type=jnp.float32,
                        precision=hi) + beff[None, :]
    h = jnp.maximum(h, 0.0)
    logits = lax.dot_general(h, w2_ref[...], (((1,), (1,)), ((), ())),
                             preferred_element_type=jnp.float32,
                             precision=hi)
    logits = logits + b2_ref[0, :][None, :]
    m = jnp.max(logits, axis=1, keepdims=True)
    e = jnp.exp(logits - m)
    alpha_ref[...] = e / jnp.sum(e, axis=1, keepdims=True)


def _alpha(rel2, s1, s2, w1p, b1, gamma, beta, w2, b2):
    return pl.pallas_call(
        _alpha_body,
        grid=(NT4,),
        in_specs=[
            pl.BlockSpec((PTS4 * LS, XW), lambda t: (t, 0)),
            pl.BlockSpec((1, XW), lambda t: (0, 0)),
            pl.BlockSpec((XW, XW), lambda t: (0, 0)),
            pl.BlockSpec((32, XW), lambda t: (0, 0)),
            pl.BlockSpec((1, 32), lambda t: (0, 0)),
            pl.BlockSpec((1, 32), lambda t: (0, 0)),
            pl.BlockSpec((1, 32), lambda t: (0, 0)),
            pl.BlockSpec((KS, 32), lambda t: (0, 0)),
            pl.BlockSpec((1, KS), lambda t: (0, 0)),
        ],
        out_specs=pl.BlockSpec((PTS4 * LS, KS), lambda t: (t, 0)),
        out_shape=jax.ShapeDtypeStruct((TOT, KS), jnp.float32),
    )(rel2, s1, s2, w1p, b1, gamma, beta, w2, b2)


PTS5 = 256
NT5 = BS * N // PTS5


def _bmmconv_body(ld_ref, alpha_ref, cw_ref, cb_ref, out_ref):
    hi = lax.Precision.DEFAULT
    a3 = alpha_ref[...].reshape(PTS5, LS, KS)
    ld = ld_ref[...]
    g = lax.dot_general(a3, ld, (((1,), (1,)), ((0,), (0,))),
                        preferred_element_type=jnp.float32,
                        precision=hi)
    acc = jnp.zeros((PTS5, OUT_CH), jnp.float32) + cb_ref[0, :][None, :]
    for k in range(KS):
        acc = acc + lax.dot_general(g[:, k, :], cw_ref[k],
                                    (((1,), (0,)), ((), ())),
                                    preferred_element_type=jnp.float32,
                                    precision=hi)
    out_ref[...] = acc


def _bmmconv(local_d3, alpha, conv_kco, conv_b):
    return pl.pallas_call(
        _bmmconv_body,
        grid=(NT5,),
        in_specs=[
            pl.BlockSpec((PTS5, LS, IN_CH), lambda t: (t, 0, 0)),
            pl.BlockSpec((PTS5 * LS, KS), lambda t: (t, 0)),
            pl.BlockSpec((KS, IN_CH, OUT_CH), lambda t: (0, 0, 0)),
            pl.BlockSpec((1, OUT_CH), lambda t: (0, 0)),
        ],
        out_specs=pl.BlockSpec((PTS5, OUT_CH), lambda t: (t, 0)),
        out_shape=jax.ShapeDtypeStruct((BS * N, OUT_CH), jnp.float32),
    )(local_d3, alpha, conv_kco, conv_b)


def kernel(xyz, data, W1, b1, gamma, beta, W2, b2, conv_w, conv_b):
    idx = _topk(xyz)

    table_d = data.reshape(BS * N, IN_CH)
    pad = jnp.zeros((BS, N, XW - 3), jnp.float32)
    table_x = jnp.concatenate([xyz, pad], axis=-1).reshape(BS * N, XW)
    local_d, rel2 = _gather(table_d, table_x, idx.reshape(TOT))
    local_d3 = local_d.reshape(BS * N, LS, IN_CH)

    s1, s2 = _moments(rel2)

    w1p = jnp.concatenate([W1, jnp.zeros((32, XW - 3), jnp.float32)], axis=1)
    alpha = _alpha(rel2, s1, s2, w1p, b1[None, :], gamma[None, :],
                   beta[None, :], W2, b2[None, :])

    conv_kco = conv_w.transpose(2, 1, 0)
    out = _bmmconv(local_d3, alpha, conv_kco, conv_b[None, :])
    return (xyz, out.reshape(BS, N, OUT_CH))


# speedup vs baseline: 1.2485x; 1.0130x over previous
import functools

import jax
import jax.numpy as jnp
from jax import lax
from jax.experimental import pallas as pl
from jax.experimental.pallas import tpu as pltpu
from jax.experimental.pallas import tpu_sc as plsc

BS = 4
N = 2048
LS = 32
KS = 16
IN_CH = 128
OUT_CH = 128
XW = 16
NBR = LS - 1
TOT = BS * N * LS

ROWS1 = 256


def _topk_body(xyz_all_ref, q_ref, idx_ref):
    b = pl.program_id(0)
    t = pl.program_id(1)
    x = xyz_all_ref[...]
    q = q_ref[...]
    qn = jnp.sum(q * q, axis=-1)
    xn = jnp.sum(x * x, axis=-1)
    qx = lax.dot_general(q, x, (((1,), (1,)), ((), ())),
                         preferred_element_type=jnp.float32)
    d = qn[:, None] + xn[None, :] - 2.0 * qx
    d = jnp.where(d < 1e-8, jnp.inf, d)
    colf = lax.broadcasted_iota(jnp.int32, (ROWS1, N), 1).astype(jnp.float32)
    base = b * N
    row0 = t * ROWS1
    self_idx = base + row0 + lax.broadcasted_iota(jnp.int32, (ROWS1,), 0)
    idx_ref[:, 0] = self_idx
    nf = jnp.float32(N)
    for k in range(NBR):
        m = jnp.min(d, axis=1, keepdims=True)
        amf = jnp.min(jnp.where(d == m, colf, nf), axis=1)
        idx_ref[:, k + 1] = amf.astype(jnp.int32) + base
        d = jnp.where(colf == amf[:, None], jnp.inf, d)


def _topk(xyz):
    return pl.pallas_call(
        _topk_body,
        grid=(BS, N // ROWS1),
        in_specs=[
            pl.BlockSpec((None, N, 3), lambda b, t: (b, 0, 0)),
            pl.BlockSpec((None, ROWS1, 3), lambda b, t: (b, t, 0)),
        ],
        out_specs=pl.BlockSpec((None, ROWS1, LS), lambda b, t: (b, t, 0)),
        out_shape=jax.ShapeDtypeStruct((BS, N, LS), jnp.int32),
    )(xyz, xyz)


NC = 2
NS = 16
NW = NC * NS
PER_W = TOT // NW
CH = 512
NCHUNK = PER_W // CH


def _gather_body(table_d, table_x, idx_hbm, out_d, out_x,
                 ia, ib, da, db, xa, xb,
                 gd0, gd1, gx0, gx1, wd0, wd1, wx0, wx1):
    wid = lax.axis_index("s") * NC + lax.axis_index("c")
    base = wid * PER_W
    idxb = [ia, ib]
    dbuf = [da, db]
    xbuf = [xa, xb]
    gds = [gd0, gd1]
    gxs = [gx0, gx1]
    wds = [wd0, wd1]
    wxs = [wx0, wx1]

    def point_rel_all(buf):
        def point_rel(p, c2):
            r0 = p * LS
            self_v = buf[r0, :]
            for l in range(1, LS):
                buf[r0 + l, :] = buf[r0 + l, :] - self_v
            buf[r0, :] = self_v - self_v
            return c2

        lax.fori_loop(0, CH // LS, point_rel, 0)

    def chunk(ci, carry):
        off = base + ci * CH
        pltpu.sync_copy(idx_hbm.at[pl.ds(off, CH)], ia)
        cp_d = pltpu.async_copy(table_d.at[ia], da, gd0)
        cp_x = pltpu.async_copy(table_x.at[ia], xa, gx0)
        cp_x.wait()
        point_rel_all(xa)
        cp_d.wait()
        pltpu.sync_copy(da, out_d.at[pl.ds(off, CH)])
        pltpu.sync_copy(xa, out_x.at[pl.ds(off, CH)])
        return carry

    lax.fori_loop(0, NCHUNK, chunk, 0)


def _gather(table_d, table_x, idx_flat):
    mesh = plsc.VectorSubcoreMesh(core_axis_name="c", subcore_axis_name="s")
    k = functools.partial(
        pl.kernel,
        mesh=mesh,
        compiler_params=pltpu.CompilerParams(use_tc_tiling_on_sc=False),
        out_type=[
            jax.ShapeDtypeStruct((TOT, IN_CH), jnp.float32),
            jax.ShapeDtypeStruct((TOT, XW), jnp.float32),
        ],
        scratch_types=[
            pltpu.VMEM((CH,), jnp.int32),
            pltpu.VMEM((CH,), jnp.int32),
            pltpu.VMEM((CH, IN_CH), jnp.float32),
            pltpu.VMEM((CH, IN_CH), jnp.float32),
            pltpu.VMEM((CH, XW), jnp.float32),
            pltpu.VMEM((CH, XW), jnp.float32),
        ] + [pltpu.SemaphoreType.DMA] * 8,
    )(_gather_body)
    return k(table_d, table_x, idx_flat)


PTS = 256
NT = BS * N // PTS
NSTEPS = 2 * NT
RPT = PTS * LS


def _k345_body(rel_ref, ld_ref, w1_ref, b1_ref, g_ref, be_ref,
               w2_ref, b2_ref, cw_ref, cb_ref, out_ref, s1_s, s2_s):
    s = pl.program_id(0)
    df = lax.Precision.DEFAULT

    @pl.when(s < NT)
    def _():
        r2 = rel_ref[...]
        p1 = jnp.sum(r2, axis=0)[None, :]
        p2 = lax.dot_general(r2, r2, (((0,), (0,)), ((), ())),
                             preferred_element_type=jnp.float32,
                             precision=lax.Precision.HIGHEST)

        @pl.when(s == 0)
        def _():
            s1_s[...] = p1
            s2_s[...] = p2

        @pl.when(s != 0)
        def _():
            s1_s[...] += p1
            s2_s[...] += p2

    @pl.when(s >= NT)
    def _():
        ntot = jnp.float32(TOT)
        mu = s1_s[0, :] / ntot
        m2 = s2_s[...] / ntot - mu[:, None] * mu[None, :]
        w1 = w1_ref[...]
        mean_h = jnp.sum(w1 * mu[None, :], axis=1) + b1_ref[0, :]
        wm = lax.dot_general(w1, m2, (((1,), (0,)), ((), ())),
                             preferred_element_type=jnp.float32,
                             precision=lax.Precision.HIGHEST)
        var_h = jnp.sum(wm * w1, axis=1)
        scale = g_ref[0, :] / jnp.sqrt(var_h + 1e-5)
        beff = (b1_ref[0, :] - mean_h) * scale + be_ref[0, :]
        weff = w1 * scale[:, None]

        rel = rel_ref[...]
        h = lax.dot_general(rel, weff, (((1,), (1,)), ((), ())),
                            preferred_element_type=jnp.float32,
                            precision=df) + beff[None, :]
        h = jnp.maximum(h, 0.0)
        logits = lax.dot_general(h, w2_ref[...], (((1,), (1,)), ((), ())),
                                 preferred_element_type=jnp.float32,
                                 precision=df)
        logits = logits + b2_ref[0, :][None, :]
        mx = jnp.max(logits, axis=1, keepdims=True)
        e = jnp.exp(logits - mx)
        alpha = e / jnp.sum(e, axis=1, keepdims=True)

        a3 = alpha.reshape(PTS, LS, KS)
        ld = ld_ref[...]
        g = lax.dot_general(a3, ld, (((1,), (1,)), ((0,), (0,))),
                            preferred_element_type=jnp.float32,
                            precision=df)
        acc = jnp.zeros((PTS, OUT_CH), jnp.float32) + cb_ref[0, :][None, :]
        for k in range(KS):
            acc = acc + lax.dot_general(g[:, k, :], cw_ref[k],
                                        (((1,), (0,)), ((), ())),
                                        preferred_element_type=jnp.float32,
                                        precision=df)
        out_ref[...] = acc


def _k345(rel2, local_d3, w1p, b1, gamma, beta, w2, b2, conv_kco, conv_b):
    def rel_map(s):
        return (jnp.where(s < NT, s, s - NT), 0)

    def ld_map(s):
        return (jnp.where(s < NT, 0, s - NT), 0, 0)

    def out_map(s):
        return (jnp.where(s < NT, 0, s - NT), 0)

    return pl.pallas_call(
        _k345_body,
        grid=(NSTEPS,),
        in_specs=[
            pl.BlockSpec((RPT, XW), rel_map),
            pl.BlockSpec((PTS, LS, IN_CH), ld_map),
            pl.BlockSpec((32, XW), lambda s: (0, 0)),
            pl.BlockSpec((1, 32), lambda s: (0, 0)),
            pl.BlockSpec((1, 32), lambda s: (0, 0)),
            pl.BlockSpec((1, 32), lambda s: (0, 0)),
            pl.BlockSpec((KS, 32), lambda s: (0, 0)),
            pl.BlockSpec((1, KS), lambda s: (0, 0)),
            pl.BlockSpec((KS, IN_CH, OUT_CH), lambda s: (0, 0, 0)),
            pl.BlockSpec((1, OUT_CH), lambda s: (0, 0)),
        ],
        out_specs=pl.BlockSpec((PTS, OUT_CH), out_map),
        out_shape=jax.ShapeDtypeStruct((BS * N, OUT_CH), jnp.float32),
        scratch_shapes=[
            pltpu.VMEM((1, XW), jnp.float32),
            pltpu.VMEM((XW, XW), jnp.float32),
        ],
    )(rel2, local_d3, w1p, b1, gamma, beta, w2, b2, conv_kco, conv_b)


def kernel(xyz, data, W1, b1, gamma, beta, W2, b2, conv_w, conv_b):
    idx = _topk(xyz)

    table_d = data.reshape(BS * N, IN_CH)
    pad = jnp.zeros((BS, N, XW - 3), jnp.float32)
    table_x = jnp.concatenate([xyz, pad], axis=-1).reshape(BS * N, XW)
    local_d, rel2 = _gather(table_d, table_x, idx.reshape(TOT))
    local_d3 = local_d.reshape(BS * N, LS, IN_CH)

    w1p = jnp.concatenate([W1, jnp.zeros((32, XW - 3), jnp.float32)], axis=1)
    conv_kco = conv_w.transpose(2, 1, 0)
    out = _k345(rel2, local_d3, w1p, b1[None, :], gamma[None, :],
                beta[None, :], W2, b2[None, :], conv_kco, conv_b[None, :])
    return (xyz, out.reshape(BS, N, OUT_CH))


# speedup vs baseline: 1.2665x; 1.0144x over previous
import functools

import jax
import jax.numpy as jnp
from jax import lax
from jax.experimental import pallas as pl
from jax.experimental.pallas import tpu as pltpu
from jax.experimental.pallas import tpu_sc as plsc

BS = 4
N = 2048
LS = 32
KS = 16
IN_CH = 128
OUT_CH = 128
XW = 16
NBR = LS - 1
TOT = BS * N * LS

ROWS1 = 512


def _topk_body(xyz_all_ref, q_ref, idx_ref):
    b = pl.program_id(0)
    t = pl.program_id(1)
    x = xyz_all_ref[...]
    q = q_ref[...]
    qn = jnp.sum(q * q, axis=-1)
    xn = jnp.sum(x * x, axis=-1)
    qx = lax.dot_general(q, x, (((1,), (1,)), ((), ())),
                         preferred_element_type=jnp.float32)
    d = qn[:, None] + xn[None, :] - 2.0 * qx
    d = jnp.where(d < 1e-8, jnp.inf, d)
    colf = lax.broadcasted_iota(jnp.int32, (ROWS1, N), 1).astype(jnp.float32)
    base = b * N
    row0 = t * ROWS1
    self_idx = base + row0 + lax.broadcasted_iota(jnp.int32, (ROWS1,), 0)
    idx_ref[:, 0] = self_idx
    nf = jnp.float32(N)
    for k in range(NBR):
        m = jnp.min(d, axis=1, keepdims=True)
        amf = jnp.min(jnp.where(d == m, colf, nf), axis=1)
        idx_ref[:, k + 1] = amf.astype(jnp.int32) + base
        d = jnp.where(colf == amf[:, None], jnp.inf, d)


def _topk(xyz):
    return pl.pallas_call(
        _topk_body,
        grid=(BS, N // ROWS1),
        in_specs=[
            pl.BlockSpec((None, N, 3), lambda b, t: (b, 0, 0)),
            pl.BlockSpec((None, ROWS1, 3), lambda b, t: (b, t, 0)),
        ],
        out_specs=pl.BlockSpec((None, ROWS1, LS), lambda b, t: (b, t, 0)),
        out_shape=jax.ShapeDtypeStruct((BS, N, LS), jnp.int32),
    )(xyz, xyz)


NC = 2
NS = 16
NW = NC * NS
PER_W = TOT // NW
CH = 256
NCHUNK = PER_W // CH


def _gather_body(table_d, table_x, idx_hbm, out_d, out_x,
                 ia, ib, da, db, xa, xb,
                 gd0, gd1, gx0, gx1, wd0, wd1, wx0, wx1):
    wid = lax.axis_index("s") * NC + lax.axis_index("c")
    base = wid * PER_W
    idxb = [ia, ib]
    dbuf = [da, db]
    xbuf = [xa, xb]
    gds = [gd0, gd1]
    gxs = [gx0, gx1]
    wds = [wd0, wd1]
    wxs = [wx0, wx1]

    def point_rel_all(buf):
        def point_rel(p, c2):
            r0 = p * LS
            self_v = buf[r0, :]
            for l in range(1, LS):
                buf[r0 + l, :] = buf[r0 + l, :] - self_v
            buf[r0, :] = self_v - self_v
            return c2

        lax.fori_loop(0, CH // LS, point_rel, 0)

    ghd = [None] * NCHUNK
    ghx = [None] * NCHUNK
    whd = [None] * NCHUNK
    whx = [None] * NCHUNK
    pltpu.sync_copy(idx_hbm.at[pl.ds(base, CH)], ia)
    ghd[0] = pltpu.async_copy(table_d.at[ia], da, gd0)
    ghx[0] = pltpu.async_copy(table_x.at[ia], xa, gx0)
    for c in range(NCHUNK):
        cb = c % 2
        nb = (c + 1) % 2
        if c + 1 < NCHUNK:
            if c >= 1:
                whd[c - 1].wait()
                whx[c - 1].wait()
            off_n = base + (c + 1) * CH
            pltpu.sync_copy(idx_hbm.at[pl.ds(off_n, CH)], idxb[nb])
            ghd[c + 1] = pltpu.async_copy(table_d.at[idxb[nb]], dbuf[nb],
                                          gds[nb])
            ghx[c + 1] = pltpu.async_copy(table_x.at[idxb[nb]], xbuf[nb],
                                          gxs[nb])
        ghx[c].wait()
        point_rel_all(xbuf[cb])
        ghd[c].wait()
        off = base + c * CH
        whd[c] = pltpu.async_copy(dbuf[cb], out_d.at[pl.ds(off, CH)],
                                  wds[cb])
        whx[c] = pltpu.async_copy(xbuf[cb], out_x.at[pl.ds(off, CH)],
                                  wxs[cb])
    whd[NCHUNK - 2].wait()
    whx[NCHUNK - 2].wait()
    whd[NCHUNK - 1].wait()
    whx[NCHUNK - 1].wait()


def _gather(table_d, table_x, idx_flat):
    mesh = plsc.VectorSubcoreMesh(core_axis_name="c", subcore_axis_name="s")
    k = functools.partial(
        pl.kernel,
        mesh=mesh,
        compiler_params=pltpu.CompilerParams(use_tc_tiling_on_sc=False),
        out_type=[
            jax.ShapeDtypeStruct((TOT, IN_CH), jnp.float32),
            jax.ShapeDtypeStruct((TOT, XW), jnp.float32),
        ],
        scratch_types=[
            pltpu.VMEM((CH,), jnp.int32),
            pltpu.VMEM((CH,), jnp.int32),
            pltpu.VMEM((CH, IN_CH), jnp.float32),
            pltpu.VMEM((CH, IN_CH), jnp.float32),
            pltpu.VMEM((CH, XW), jnp.float32),
            pltpu.VMEM((CH, XW), jnp.float32),
        ] + [pltpu.SemaphoreType.DMA] * 8,
    )(_gather_body)
    return k(table_d, table_x, idx_flat)


PTS = 256
NT = BS * N // PTS
NSTEPS = 2 * NT
RPT = PTS * LS


def _k345_body(rel_ref, ld_ref, w1_ref, b1_ref, g_ref, be_ref,
               w2_ref, b2_ref, cw_ref, cb_ref, out_ref, s1_s, s2_s):
    s = pl.program_id(0)
    df = lax.Precision.DEFAULT

    @pl.when(s < NT)
    def _():
        r2 = rel_ref[...]
        p1 = jnp.sum(r2, axis=0)[None, :]
        p2 = lax.dot_general(r2, r2, (((0,), (0,)), ((), ())),
                             preferred_element_type=jnp.float32,
                             precision=lax.Precision.HIGHEST)

        @pl.when(s == 0)
        def _():
            s1_s[...] = p1
            s2_s[...] = p2

        @pl.when(s != 0)
        def _():
            s1_s[...] += p1
            s2_s[...] += p2

    @pl.when(s >= NT)
    def _():
        ntot = jnp.float32(TOT)
        mu = s1_s[0, :] / ntot
        m2 = s2_s[...] / ntot - mu[:, None] * mu[None, :]
        w1 = w1_ref[...]
        mean_h = jnp.sum(w1 * mu[None, :], axis=1) + b1_ref[0, :]
        wm = lax.dot_general(w1, m2, (((1,), (0,)), ((), ())),
                             preferred_element_type=jnp.float32,
                             precision=lax.Precision.HIGHEST)
        var_h = jnp.sum(wm * w1, axis=1)
        scale = g_ref[0, :] / jnp.sqrt(var_h + 1e-5)
        beff = (b1_ref[0, :] - mean_h) * scale + be_ref[0, :]
        weff = w1 * scale[:, None]

        rel = rel_ref[...]
        h = lax.dot_general(rel, weff, (((1,), (1,)), ((), ())),
                            preferred_element_type=jnp.float32,
                            precision=df) + beff[None, :]
        h = jnp.maximum(h, 0.0)
        logits = lax.dot_general(h, w2_ref[...], (((1,), (1,)), ((), ())),
                                 preferred_element_type=jnp.float32,
                                 precision=df)
        logits = logits + b2_ref[0, :][None, :]
        mx = jnp.max(logits, axis=1, keepdims=True)
        e = jnp.exp(logits - mx)
        alpha = e / jnp.sum(e, axis=1, keepdims=True)

        a3 = alpha.reshape(PTS, LS, KS)
        ld = ld_ref[...]
        g = lax.dot_general(a3, ld, (((1,), (1,)), ((0,), (0,))),
                            preferred_element_type=jnp.float32,
                            precision=df)
        acc = jnp.zeros((PTS, OUT_CH), jnp.float32) + cb_ref[0, :][None, :]
        for k in range(KS):
            acc = acc + lax.dot_general(g[:, k, :], cw_ref[k],
                                        (((1,), (0,)), ((), ())),
                                        preferred_element_type=jnp.float32,
                                        precision=df)
        out_ref[...] = acc


def _k345(rel2, local_d3, w1p, b1, gamma, beta, w2, b2, conv_kco, conv_b):
    def rel_map(s):
        return (jnp.where(s < NT, s, s - NT), 0)

    def ld_map(s):
        return (jnp.where(s < NT, 0, s - NT), 0, 0)

    def out_map(s):
        return (jnp.where(s < NT, 0, s - NT), 0)

    return pl.pallas_call(
        _k345_body,
        grid=(NSTEPS,),
        in_specs=[
            pl.BlockSpec((RPT, XW), rel_map),
            pl.BlockSpec((PTS, LS, IN_CH), ld_map),
            pl.BlockSpec((32, XW), lambda s: (0, 0)),
            pl.BlockSpec((1, 32), lambda s: (0, 0)),
            pl.BlockSpec((1, 32), lambda s: (0, 0)),
            pl.BlockSpec((1, 32), lambda s: (0, 0)),
            pl.BlockSpec((KS, 32), lambda s: (0, 0)),
            pl.BlockSpec((1, KS), lambda s: (0, 0)),
            pl.BlockSpec((KS, IN_CH, OUT_CH), lambda s: (0, 0, 0)),
            pl.BlockSpec((1, OUT_CH), lambda s: (0, 0)),
        ],
        out_specs=pl.BlockSpec((PTS, OUT_CH), out_map),
        out_shape=jax.ShapeDtypeStruct((BS * N, OUT_CH), jnp.float32),
        scratch_shapes=[
            pltpu.VMEM((1, XW), jnp.float32),
            pltpu.VMEM((XW, XW), jnp.float32),
        ],
    )(rel2, local_d3, w1p, b1, gamma, beta, w2, b2, conv_kco, conv_b)


def kernel(xyz, data, W1, b1, gamma, beta, W2, b2, conv_w, conv_b):
    idx = _topk(xyz)

    table_d = data.reshape(BS * N, IN_CH)
    pad = jnp.zeros((BS, N, XW - 3), jnp.float32)
    table_x = jnp.concatenate([xyz, pad], axis=-1).reshape(BS * N, XW)
    local_d, rel2 = _gather(table_d, table_x, idx.reshape(TOT))
    local_d3 = local_d.reshape(BS * N, LS, IN_CH)

    w1p = jnp.concatenate([W1, jnp.zeros((32, XW - 3), jnp.float32)], axis=1)
    conv_kco = conv_w.transpose(2, 1, 0)
    out = _k345(rel2, local_d3, w1p, b1[None, :], gamma[None, :],
                beta[None, :], W2, b2[None, :], conv_kco, conv_b[None, :])
    return (xyz, out.reshape(BS, N, OUT_CH))
